# baseline (speedup 1.0000x reference)
import functools

import jax
import jax.numpy as jnp
from jax import lax
from jax.experimental import pallas as pl
from jax.experimental.pallas import tpu as pltpu
from jax.experimental.pallas import tpu_sc as plsc

N = 10000
E = 320000
D = 128
C = 16
G = 64
BN_EPS = 1e-5

NC = 2
NS = 16
NW = NC * NS

CH = 80
KC = 128
E_PAD = NW * KC * CH
NBUF = 4
NHALF = 4
CPH = KC // NHALF

RPT = 632
N_PAD = RPT * NS

R = 1000
NB = N // R


def _segsum_sc(h, srcw, dstw, zpad):
  mesh = plsc.VectorSubcoreMesh(
      core_axis_name="c", subcore_axis_name="s", num_cores=NC,
      num_subcores=NS)

  @functools.partial(
      pl.kernel,
      out_type=jax.ShapeDtypeStruct((NC, N_PAD, D), jnp.float32),
      mesh=mesh,
      scratch_types=[
          pltpu.VMEM((CPH, CH), jnp.int32),
          pltpu.VMEM((CPH, CH), jnp.int32),
          pltpu.VMEM((NBUF, CH, D), jnp.float32),
          pltpu.VMEM_SHARED((N_PAD, D), jnp.float32),
          pltpu.SemaphoreType.DMA((NBUF,)),
          pltpu.SemaphoreType.DMA,
      ])
  def k(h_hbm, src_hbm, dst_hbm, z_hbm, out_hbm, src_v, dst_v, rows_v,
        acc_sh, gsems, isem):
    cid = lax.axis_index("c")
    sid = lax.axis_index("s")
    wid = sid * NC + cid
    NF = (N - (NS - 1) * RPT)

    @pl.when(cid == 0)
    def _():
      @pl.when(sid < NS - 1)
      def _():
        pltpu.async_copy(h_hbm.at[pl.ds(sid * RPT, RPT)],
                         acc_sh.at[pl.ds(sid * RPT, RPT)], isem)

      @pl.when(sid == NS - 1)
      def _():
        pltpu.async_copy(h_hbm.at[pl.ds((NS - 1) * RPT, NF)],
                         acc_sh.at[pl.ds((NS - 1) * RPT, NF)], isem)
        pltpu.async_copy(z_hbm.at[pl.ds(0, N_PAD - N)],
                         acc_sh.at[pl.ds(N, N_PAD - N)], isem)

    @pl.when(cid == 1)
    def _():
      pltpu.async_copy(z_hbm, acc_sh.at[pl.ds(sid * RPT, RPT)], isem)

    for half in range(NHALF):
      pltpu.sync_copy(src_hbm.at[wid, pl.ds(half * CPH, CPH)], src_v)
      pltpu.sync_copy(dst_hbm.at[wid, pl.ds(half * CPH, CPH)], dst_v)
      for b in range(NBUF):
        pltpu.async_copy(h_hbm.at[src_v.at[b]], rows_v.at[b], gsems.at[b])

      if half == 0:
        @pl.when(cid == 0)
        def _():
          @pl.when(sid < NS - 1)
          def _():
            pltpu.make_async_copy(h_hbm.at[pl.ds(sid * RPT, RPT)],
                                  acc_sh.at[pl.ds(sid * RPT, RPT)],
                                  isem).wait()

          @pl.when(sid == NS - 1)
          def _():
            pltpu.make_async_copy(h_hbm.at[pl.ds((NS - 1) * RPT, NF)],
                                  acc_sh.at[pl.ds((NS - 1) * RPT, NF)],
                                  isem).wait()
            pltpu.make_async_copy(z_hbm.at[pl.ds(0, N_PAD - N)],
                                  acc_sh.at[pl.ds(N, N_PAD - N)],
                                  isem).wait()

        @pl.when(cid == 1)
        def _():
          pltpu.make_async_copy(z_hbm, acc_sh.at[pl.ds(sid * RPT, RPT)],
                                isem).wait()

        plsc.subcore_barrier()

      @pl.loop(0, CPH // NBUF)
      def body(gq):
        for b in range(NBUF):
          g = gq * NBUF + b
          pltpu.make_async_copy(h_hbm.at[src_v.at[0]], rows_v.at[b],
                                gsems.at[b]).wait()
          pltpu.sync_copy(rows_v.at[b], acc_sh.at[dst_v.at[g]], add=True)

          @pl.when(gq < CPH // NBUF - 1)
          def _():
            pltpu.async_copy(h_hbm.at[src_v.at[g + NBUF]], rows_v.at[b],
                             gsems.at[b])

    plsc.subcore_barrier()
    pltpu.sync_copy(acc_sh.at[pl.ds(sid * RPT, RPT)],
                    out_hbm.at[cid, pl.ds(sid * RPT, RPT)])

  return k(h, srcw, dstw, zpad)


def _gin_layer_tc(agg, W1, b1, W2, b2, gamma, beta, batch3=None,
                  l1w=None, l1b=None, l2w=None, l2b=None):
  final = batch3 is not None

  def body(*refs):
    if final:
      (agg_ref, w1_ref, b1_ref, w2_ref, b2_ref, g_ref, be_ref,
       bt_ref, hw1_ref, hb1_ref, hw2_ref, hb2_ref, e_ref, out_ref,
       vs_ref, st_ref, pool_ref, cnt_ref) = refs
    else:
      (agg_ref, w1_ref, b1_ref, w2_ref, b2_ref, g_ref, be_ref,
       e_ref, vs_ref, st_ref) = refs
    p = pl.program_id(0)
    i = pl.program_id(1)

    @pl.when(p == 0)
    def _():
      @pl.when(i == 0)
      def _():
        st_ref[...] = jnp.zeros_like(st_ref)

      z = agg_ref[0] + agg_ref[1]
      u = jnp.maximum(jnp.dot(z, w1_ref[...],
                              preferred_element_type=jnp.float32)
                      + b1_ref[...], 0.0)
      v = jnp.maximum(jnp.dot(u, w2_ref[...],
                              preferred_element_type=jnp.float32)
                      + b2_ref[...], 0.0)
      vs_ref[i] = v
      st_ref[0:1, :] += jnp.sum(v, axis=0, keepdims=True)
      st_ref[1:2, :] += jnp.sum(v * v, axis=0, keepdims=True)

    @pl.when(p == 1)
    def _():
      mean = st_ref[0:1, :] * (1.0 / N)
      var = st_ref[1:2, :] * (1.0 / N) - mean * mean
      scale = g_ref[...] * lax.rsqrt(var + BN_EPS)
      e = (vs_ref[i] - mean) * scale + be_ref[...]
      e_ref[...] = e
      if final:
        @pl.when(i == 0)
        def _():
          pool_ref[...] = jnp.zeros_like(pool_ref)
          cnt_ref[...] = jnp.zeros_like(cnt_ref)

        ids = bt_ref[0]
        gid = lax.broadcasted_iota(jnp.int32, (G, R), 0)
        onehot = (gid == ids).astype(jnp.float32)
        pool_ref[...] += jnp.dot(onehot, e,
                                 preferred_element_type=jnp.float32,
                                 precision=lax.Precision.HIGHEST)
        cnt_ref[...] += jnp.dot(onehot, jnp.ones((R, D), jnp.float32),
                                preferred_element_type=jnp.float32,
                                precision=lax.Precision.HIGHEST)

        @pl.when(i == NB - 1)
        def _():
          pooled = pool_ref[...] / jnp.maximum(cnt_ref[...], 1.0)
          o = jnp.dot(pooled, hw1_ref[...],
                      preferred_element_type=jnp.float32,
                      precision=lax.Precision.HIGHEST) + hb1_ref[...]
          o = jnp.dot(o, hw2_ref[...],
                      preferred_element_type=jnp.float32,
                      precision=lax.Precision.HIGHEST) + hb2_ref[...]
          out_ref[...] = jnp.clip(o, -10.0, 10.0)

  const = lambda pp, ii: (0, 0)
  in_specs = [
      pl.BlockSpec((NC, R, D), lambda p, i: (0, (1 - p) * i, 0)),
      pl.BlockSpec((D, D), const),
      pl.BlockSpec((1, D), const),
      pl.BlockSpec((D, D), const),
      pl.BlockSpec((1, D), const),
      pl.BlockSpec((1, D), const),
      pl.BlockSpec((1, D), const),
  ]
  out_specs = [pl.BlockSpec((R, D), lambda p, i: (p * i, 0))]
  out_shape = [jax.ShapeDtypeStruct((N, D), jnp.float32)]
  scratch = [
      pltpu.VMEM((NB, R, D), jnp.float32),
      pltpu.VMEM((8, D), jnp.float32),
  ]
  args = [agg, W1, b1, W2, b2, gamma, beta]
  if final:
    in_specs += [
        pl.BlockSpec((1, 1, R), lambda p, i: (p * i, 0, 0)),
        pl.BlockSpec((D, D), const),
        pl.BlockSpec((1, D), const),
        pl.BlockSpec((D, D), const),
        pl.BlockSpec((1, D), const),
    ]
    out_specs += [pl.BlockSpec((G, D), const)]
    out_shape += [jax.ShapeDtypeStruct((G, D), jnp.float32)]
    scratch += [pltpu.VMEM((G, D), jnp.float32),
                pltpu.VMEM((G, D), jnp.float32)]
    args += [batch3, l1w, l1b, l2w, l2b]

  res = pl.pallas_call(
      body,
      grid=(2, NB),
      in_specs=in_specs,
      out_specs=out_specs,
      out_shape=out_shape,
      scratch_shapes=scratch,
  )(*args)
  return res


def kernel(x, edge_index, batch, params):
  src = edge_index[0]
  dst = edge_index[1]
  pad = E_PAD - E
  pad_src = jnp.arange(pad, dtype=jnp.int32) % N
  pad_dst = N + (jnp.arange(pad, dtype=jnp.int32) % (N_PAD - N))
  srcw = jnp.concatenate([src, pad_src]).reshape(NW, KC, CH)
  dstw = jnp.concatenate([dst, pad_dst]).reshape(NW, KC, CH)
  zpad = jnp.zeros((RPT, D), jnp.float32)
  batch3 = batch.reshape(NB, 1, R)

  row = lambda a: a.reshape(1, D)
  l2w = jnp.zeros((D, D), jnp.float32).at[:, :C].set(params["lin2_W"])
  l2b = jnp.zeros((1, D), jnp.float32).at[0, :C].set(params["lin2_b"])

  h = x
  for li in range(len(params["convs"])):
    p = params["convs"][li]
    agg = _segsum_sc(h, srcw, dstw, zpad)
    if li < len(params["convs"]) - 1:
      (h,) = _gin_layer_tc(agg, p["W1"], row(p["b1"]), p["W2"],
                           row(p["b2"]), row(p["gamma"]), row(p["beta"]))
    else:
      embeds, out_pad = _gin_layer_tc(
          agg, p["W1"], row(p["b1"]), p["W2"], row(p["b2"]),
          row(p["gamma"]), row(p["beta"]), batch3,
          params["lin1_W"], row(params["lin1_b"]), l2w, l2b)
  return (out_pad[:, :C], embeds)


# speedup vs baseline: 1.0466x; 1.0466x over previous
import functools

import jax
import jax.numpy as jnp
from jax import lax
from jax.experimental import pallas as pl
from jax.experimental.pallas import tpu as pltpu
from jax.experimental.pallas import tpu_sc as plsc

N = 10000
E = 320000
D = 128
C = 16
G = 64
BN_EPS = 1e-5

NC = 2
NS = 16
NW = NC * NS

CH = 80
KC = 128
E_PAD = NW * KC * CH
NBUF = 4
NHALF = 4
CPH = KC // NHALF

RPT = 632
N_PAD = RPT * NS

R = 5000
NB = N // R


def _segsum_sc(h, srcw, dstw, zpad):
  mesh = plsc.VectorSubcoreMesh(
      core_axis_name="c", subcore_axis_name="s", num_cores=NC,
      num_subcores=NS)

  @functools.partial(
      pl.kernel,
      out_type=jax.ShapeDtypeStruct((NC, N_PAD, D), jnp.float32),
      mesh=mesh,
      scratch_types=[
          pltpu.VMEM((CPH, CH), jnp.int32),
          pltpu.VMEM((CPH, CH), jnp.int32),
          pltpu.VMEM((NBUF, CH, D), jnp.float32),
          pltpu.VMEM_SHARED((N_PAD, D), jnp.float32),
          pltpu.SemaphoreType.DMA((NBUF,)),
          pltpu.SemaphoreType.DMA,
      ])
  def k(h_hbm, src_hbm, dst_hbm, z_hbm, out_hbm, src_v, dst_v, rows_v,
        acc_sh, gsems, isem):
    cid = lax.axis_index("c")
    sid = lax.axis_index("s")
    wid = sid * NC + cid
    NF = (N - (NS - 1) * RPT)

    @pl.when(cid == 0)
    def _():
      @pl.when(sid < NS - 1)
      def _():
        pltpu.async_copy(h_hbm.at[pl.ds(sid * RPT, RPT)],
                         acc_sh.at[pl.ds(sid * RPT, RPT)], isem)

      @pl.when(sid == NS - 1)
      def _():
        pltpu.async_copy(h_hbm.at[pl.ds((NS - 1) * RPT, NF)],
                         acc_sh.at[pl.ds((NS - 1) * RPT, NF)], isem)
        pltpu.async_copy(z_hbm.at[pl.ds(0, N_PAD - N)],
                         acc_sh.at[pl.ds(N, N_PAD - N)], isem)

    @pl.when(cid == 1)
    def _():
      pltpu.async_copy(z_hbm, acc_sh.at[pl.ds(sid * RPT, RPT)], isem)

    for half in range(NHALF):
      pltpu.sync_copy(src_hbm.at[wid, pl.ds(half * CPH, CPH)], src_v)
      pltpu.sync_copy(dst_hbm.at[wid, pl.ds(half * CPH, CPH)], dst_v)
      for b in range(NBUF):
        pltpu.async_copy(h_hbm.at[src_v.at[b]], rows_v.at[b], gsems.at[b])

      if half == 0:
        @pl.when(cid == 0)
        def _():
          @pl.when(sid < NS - 1)
          def _():
            pltpu.make_async_copy(h_hbm.at[pl.ds(sid * RPT, RPT)],
                                  acc_sh.at[pl.ds(sid * RPT, RPT)],
                                  isem).wait()

          @pl.when(sid == NS - 1)
          def _():
            pltpu.make_async_copy(h_hbm.at[pl.ds((NS - 1) * RPT, NF)],
                                  acc_sh.at[pl.ds((NS - 1) * RPT, NF)],
                                  isem).wait()
            pltpu.make_async_copy(z_hbm.at[pl.ds(0, N_PAD - N)],
                                  acc_sh.at[pl.ds(N, N_PAD - N)],
                                  isem).wait()

        @pl.when(cid == 1)
        def _():
          pltpu.make_async_copy(z_hbm, acc_sh.at[pl.ds(sid * RPT, RPT)],
                                isem).wait()

        plsc.subcore_barrier()

      @pl.loop(0, CPH // NBUF)
      def body(gq):
        for b in range(NBUF):
          g = gq * NBUF + b
          pltpu.make_async_copy(h_hbm.at[src_v.at[0]], rows_v.at[b],
                                gsems.at[b]).wait()
          pltpu.sync_copy(rows_v.at[b], acc_sh.at[dst_v.at[g]], add=True)

          @pl.when(gq < CPH // NBUF - 1)
          def _():
            pltpu.async_copy(h_hbm.at[src_v.at[g + NBUF]], rows_v.at[b],
                             gsems.at[b])

    plsc.subcore_barrier()
    pltpu.sync_copy(acc_sh.at[pl.ds(sid * RPT, RPT)],
                    out_hbm.at[cid, pl.ds(sid * RPT, RPT)])

  return k(h, srcw, dstw, zpad)


def _gin_layer_tc(agg, W1, b1, W2, b2, gamma, beta, batch3=None,
                  l1w=None, l1b=None, l2w=None, l2b=None):
  final = batch3 is not None

  def body(*refs):
    if final:
      (agg_ref, w1_ref, b1_ref, w2_ref, b2_ref, g_ref, be_ref,
       bt_ref, hw1_ref, hb1_ref, hw2_ref, hb2_ref, e_ref, out_ref,
       vs_ref, st_ref, pool_ref, cnt_ref) = refs
    else:
      (agg_ref, w1_ref, b1_ref, w2_ref, b2_ref, g_ref, be_ref,
       e_ref, vs_ref, st_ref) = refs
    p = pl.program_id(0)
    i = pl.program_id(1)

    @pl.when(p == 0)
    def _():
      @pl.when(i == 0)
      def _():
        st_ref[...] = jnp.zeros_like(st_ref)

      z = agg_ref[0] + agg_ref[1]
      u = jnp.maximum(jnp.dot(z, w1_ref[...],
                              preferred_element_type=jnp.float32)
                      + b1_ref[...], 0.0)
      v = jnp.maximum(jnp.dot(u, w2_ref[...],
                              preferred_element_type=jnp.float32)
                      + b2_ref[...], 0.0)
      vs_ref[i] = v
      st_ref[0:1, :] += jnp.sum(v, axis=0, keepdims=True)
      st_ref[1:2, :] += jnp.sum(v * v, axis=0, keepdims=True)

    @pl.when(p == 1)
    def _():
      mean = st_ref[0:1, :] * (1.0 / N)
      var = st_ref[1:2, :] * (1.0 / N) - mean * mean
      scale = g_ref[...] * lax.rsqrt(var + BN_EPS)
      e = (vs_ref[i] - mean) * scale + be_ref[...]
      e_ref[...] = e
      if final:
        @pl.when(i == 0)
        def _():
          pool_ref[...] = jnp.zeros_like(pool_ref)
          cnt_ref[...] = jnp.zeros_like(cnt_ref)

        ids = bt_ref[0]
        gid = lax.broadcasted_iota(jnp.int32, (G, R), 0)
        onehot = (gid == ids).astype(jnp.float32)
        pool_ref[...] += jnp.dot(onehot, e,
                                 preferred_element_type=jnp.float32,
                                 precision=lax.Precision.HIGHEST)
        cnt_ref[...] += jnp.dot(onehot, jnp.ones((R, D), jnp.float32),
                                preferred_element_type=jnp.float32,
                                precision=lax.Precision.HIGHEST)

        @pl.when(i == NB - 1)
        def _():
          pooled = pool_ref[...] / jnp.maximum(cnt_ref[...], 1.0)
          o = jnp.dot(pooled, hw1_ref[...],
                      preferred_element_type=jnp.float32,
                      precision=lax.Precision.HIGHEST) + hb1_ref[...]
          o = jnp.dot(o, hw2_ref[...],
                      preferred_element_type=jnp.float32,
                      precision=lax.Precision.HIGHEST) + hb2_ref[...]
          out_ref[...] = jnp.clip(o, -10.0, 10.0)

  const = lambda pp, ii: (0, 0)
  in_specs = [
      pl.BlockSpec((NC, R, D), lambda p, i: (0, (1 - p) * i, 0)),
      pl.BlockSpec((D, D), const),
      pl.BlockSpec((1, D), const),
      pl.BlockSpec((D, D), const),
      pl.BlockSpec((1, D), const),
      pl.BlockSpec((1, D), const),
      pl.BlockSpec((1, D), const),
  ]
  out_specs = [pl.BlockSpec((R, D), lambda p, i: (p * i, 0))]
  out_shape = [jax.ShapeDtypeStruct((N, D), jnp.float32)]
  scratch = [
      pltpu.VMEM((NB, R, D), jnp.float32),
      pltpu.VMEM((8, D), jnp.float32),
  ]
  args = [agg, W1, b1, W2, b2, gamma, beta]
  if final:
    in_specs += [
        pl.BlockSpec((1, 1, R), lambda p, i: (p * i, 0, 0)),
        pl.BlockSpec((D, D), const),
        pl.BlockSpec((1, D), const),
        pl.BlockSpec((D, D), const),
        pl.BlockSpec((1, D), const),
    ]
    out_specs += [pl.BlockSpec((G, D), const)]
    out_shape += [jax.ShapeDtypeStruct((G, D), jnp.float32)]
    scratch += [pltpu.VMEM((G, D), jnp.float32),
                pltpu.VMEM((G, D), jnp.float32)]
    args += [batch3, l1w, l1b, l2w, l2b]

  res = pl.pallas_call(
      body,
      grid=(2, NB),
      in_specs=in_specs,
      out_specs=out_specs,
      out_shape=out_shape,
      scratch_shapes=scratch,
  )(*args)
  return res


def kernel(x, edge_index, batch, params):
  src = edge_index[0]
  dst = edge_index[1]
  pad = E_PAD - E
  pad_src = jnp.arange(pad, dtype=jnp.int32) % N
  pad_dst = N + (jnp.arange(pad, dtype=jnp.int32) % (N_PAD - N))
  srcw = jnp.concatenate([src, pad_src]).reshape(NW, KC, CH)
  dstw = jnp.concatenate([dst, pad_dst]).reshape(NW, KC, CH)
  zpad = jnp.zeros((RPT, D), jnp.float32)
  batch3 = batch.reshape(NB, 1, R)

  row = lambda a: a.reshape(1, D)
  l2w = jnp.zeros((D, D), jnp.float32).at[:, :C].set(params["lin2_W"])
  l2b = jnp.zeros((1, D), jnp.float32).at[0, :C].set(params["lin2_b"])

  h = x
  for li in range(len(params["convs"])):
    p = params["convs"][li]
    agg = _segsum_sc(h, srcw, dstw, zpad)
    if li < len(params["convs"]) - 1:
      (h,) = _gin_layer_tc(agg, p["W1"], row(p["b1"]), p["W2"],
                           row(p["b2"]), row(p["gamma"]), row(p["beta"]))
    else:
      embeds, out_pad = _gin_layer_tc(
          agg, p["W1"], row(p["b1"]), p["W2"], row(p["b2"]),
          row(p["gamma"]), row(p["beta"]), batch3,
          params["lin1_W"], row(params["lin1_b"]), l2w, l2b)
  return (out_pad[:, :C], embeds)


# speedup vs baseline: 1.0505x; 1.0037x over previous
import functools

import jax
import jax.numpy as jnp
from jax import lax
from jax.experimental import pallas as pl
from jax.experimental.pallas import tpu as pltpu
from jax.experimental.pallas import tpu_sc as plsc

N = 10000
E = 320000
D = 128
C = 16
G = 64
BN_EPS = 1e-5

NC = 2
NS = 16
NW = NC * NS

CH = 80
KC = 128
E_PAD = NW * KC * CH
NBUF = 4
NHALF = 4
CPH = KC // NHALF

RPT = 632
N_PAD = RPT * NS

R = 10000
NB = N // R


def _segsum_sc(h, srcw, dstw, zpad):
  mesh = plsc.VectorSubcoreMesh(
      core_axis_name="c", subcore_axis_name="s", num_cores=NC,
      num_subcores=NS)

  @functools.partial(
      pl.kernel,
      out_type=jax.ShapeDtypeStruct((NC, N_PAD, D), jnp.float32),
      mesh=mesh,
      scratch_types=[
          pltpu.VMEM((CPH, CH), jnp.int32),
          pltpu.VMEM((CPH, CH), jnp.int32),
          pltpu.VMEM((NBUF, CH, D), jnp.float32),
          pltpu.VMEM_SHARED((N_PAD, D), jnp.float32),
          pltpu.SemaphoreType.DMA((NBUF,)),
          pltpu.SemaphoreType.DMA,
      ])
  def k(h_hbm, src_hbm, dst_hbm, z_hbm, out_hbm, src_v, dst_v, rows_v,
        acc_sh, gsems, isem):
    cid = lax.axis_index("c")
    sid = lax.axis_index("s")
    wid = sid * NC + cid
    NF = (N - (NS - 1) * RPT)

    @pl.when(cid == 0)
    def _():
      @pl.when(sid < NS - 1)
      def _():
        pltpu.async_copy(h_hbm.at[pl.ds(sid * RPT, RPT)],
                         acc_sh.at[pl.ds(sid * RPT, RPT)], isem)

      @pl.when(sid == NS - 1)
      def _():
        pltpu.async_copy(h_hbm.at[pl.ds((NS - 1) * RPT, NF)],
                         acc_sh.at[pl.ds((NS - 1) * RPT, NF)], isem)
        pltpu.async_copy(z_hbm.at[pl.ds(0, N_PAD - N)],
                         acc_sh.at[pl.ds(N, N_PAD - N)], isem)

    @pl.when(cid == 1)
    def _():
      pltpu.async_copy(z_hbm, acc_sh.at[pl.ds(sid * RPT, RPT)], isem)

    for half in range(NHALF):
      pltpu.sync_copy(src_hbm.at[wid, pl.ds(half * CPH, CPH)], src_v)
      pltpu.sync_copy(dst_hbm.at[wid, pl.ds(half * CPH, CPH)], dst_v)
      for b in range(NBUF):
        pltpu.async_copy(h_hbm.at[src_v.at[b]], rows_v.at[b], gsems.at[b])

      if half == 0:
        @pl.when(cid == 0)
        def _():
          @pl.when(sid < NS - 1)
          def _():
            pltpu.make_async_copy(h_hbm.at[pl.ds(sid * RPT, RPT)],
                                  acc_sh.at[pl.ds(sid * RPT, RPT)],
                                  isem).wait()

          @pl.when(sid == NS - 1)
          def _():
            pltpu.make_async_copy(h_hbm.at[pl.ds((NS - 1) * RPT, NF)],
                                  acc_sh.at[pl.ds((NS - 1) * RPT, NF)],
                                  isem).wait()
            pltpu.make_async_copy(z_hbm.at[pl.ds(0, N_PAD - N)],
                                  acc_sh.at[pl.ds(N, N_PAD - N)],
                                  isem).wait()

        @pl.when(cid == 1)
        def _():
          pltpu.make_async_copy(z_hbm, acc_sh.at[pl.ds(sid * RPT, RPT)],
                                isem).wait()

        plsc.subcore_barrier()

      @pl.loop(0, CPH // NBUF)
      def body(gq):
        for b in range(NBUF):
          g = gq * NBUF + b
          pltpu.make_async_copy(h_hbm.at[src_v.at[0]], rows_v.at[b],
                                gsems.at[b]).wait()
          pltpu.sync_copy(rows_v.at[b], acc_sh.at[dst_v.at[g]], add=True)

          @pl.when(gq < CPH // NBUF - 1)
          def _():
            pltpu.async_copy(h_hbm.at[src_v.at[g + NBUF]], rows_v.at[b],
                             gsems.at[b])

    plsc.subcore_barrier()
    pltpu.sync_copy(acc_sh.at[pl.ds(sid * RPT, RPT)],
                    out_hbm.at[cid, pl.ds(sid * RPT, RPT)])

  return k(h, srcw, dstw, zpad)


def _gin_layer_tc(agg, W1, b1, W2, b2, gamma, beta, batch3=None,
                  l1w=None, l1b=None, l2w=None, l2b=None):
  final = batch3 is not None

  def body(*refs):
    if final:
      (agg_ref, w1_ref, b1_ref, w2_ref, b2_ref, g_ref, be_ref,
       bt_ref, hw1_ref, hb1_ref, hw2_ref, hb2_ref, e_ref, out_ref,
       vs_ref, st_ref, pool_ref, cnt_ref) = refs
    else:
      (agg_ref, w1_ref, b1_ref, w2_ref, b2_ref, g_ref, be_ref,
       e_ref, vs_ref, st_ref) = refs
    p = pl.program_id(0)
    i = pl.program_id(1)

    @pl.when(p == 0)
    def _():
      @pl.when(i == 0)
      def _():
        st_ref[...] = jnp.zeros_like(st_ref)

      z = agg_ref[0] + agg_ref[1]
      u = jnp.maximum(jnp.dot(z, w1_ref[...],
                              preferred_element_type=jnp.float32)
                      + b1_ref[...], 0.0)
      v = jnp.maximum(jnp.dot(u, w2_ref[...],
                              preferred_element_type=jnp.float32)
                      + b2_ref[...], 0.0)
      vs_ref[i] = v
      st_ref[0:1, :] += jnp.sum(v, axis=0, keepdims=True)
      st_ref[1:2, :] += jnp.sum(v * v, axis=0, keepdims=True)

    @pl.when(p == 1)
    def _():
      mean = st_ref[0:1, :] * (1.0 / N)
      var = st_ref[1:2, :] * (1.0 / N) - mean * mean
      scale = g_ref[...] * lax.rsqrt(var + BN_EPS)
      e = (vs_ref[i] - mean) * scale + be_ref[...]
      e_ref[...] = e
      if final:
        @pl.when(i == 0)
        def _():
          pool_ref[...] = jnp.zeros_like(pool_ref)
          cnt_ref[...] = jnp.zeros_like(cnt_ref)

        ids = bt_ref[0]
        gid = lax.broadcasted_iota(jnp.int32, (G, R), 0)
        onehot = (gid == ids).astype(jnp.float32)
        pool_ref[...] += jnp.dot(onehot, e,
                                 preferred_element_type=jnp.float32,
                                 precision=lax.Precision.HIGHEST)
        cnt_ref[...] += jnp.dot(onehot, jnp.ones((R, D), jnp.float32),
                                preferred_element_type=jnp.float32,
                                precision=lax.Precision.HIGHEST)

        @pl.when(i == NB - 1)
        def _():
          pooled = pool_ref[...] / jnp.maximum(cnt_ref[...], 1.0)
          o = jnp.dot(pooled, hw1_ref[...],
                      preferred_element_type=jnp.float32,
                      precision=lax.Precision.HIGHEST) + hb1_ref[...]
          o = jnp.dot(o, hw2_ref[...],
                      preferred_element_type=jnp.float32,
                      precision=lax.Precision.HIGHEST) + hb2_ref[...]
          out_ref[...] = jnp.clip(o, -10.0, 10.0)

  const = lambda pp, ii: (0, 0)
  in_specs = [
      pl.BlockSpec((NC, R, D), lambda p, i: (0, (1 - p) * i, 0)),
      pl.BlockSpec((D, D), const),
      pl.BlockSpec((1, D), const),
      pl.BlockSpec((D, D), const),
      pl.BlockSpec((1, D), const),
      pl.BlockSpec((1, D), const),
      pl.BlockSpec((1, D), const),
  ]
  out_specs = [pl.BlockSpec((R, D), lambda p, i: (p * i, 0))]
  out_shape = [jax.ShapeDtypeStruct((N, D), jnp.float32)]
  scratch = [
      pltpu.VMEM((NB, R, D), jnp.float32),
      pltpu.VMEM((8, D), jnp.float32),
  ]
  args = [agg, W1, b1, W2, b2, gamma, beta]
  if final:
    in_specs += [
        pl.BlockSpec((1, 1, R), lambda p, i: (p * i, 0, 0)),
        pl.BlockSpec((D, D), const),
        pl.BlockSpec((1, D), const),
        pl.BlockSpec((D, D), const),
        pl.BlockSpec((1, D), const),
    ]
    out_specs += [pl.BlockSpec((G, D), const)]
    out_shape += [jax.ShapeDtypeStruct((G, D), jnp.float32)]
    scratch += [pltpu.VMEM((G, D), jnp.float32),
                pltpu.VMEM((G, D), jnp.float32)]
    args += [batch3, l1w, l1b, l2w, l2b]

  res = pl.pallas_call(
      body,
      grid=(2, NB),
      in_specs=in_specs,
      out_specs=out_specs,
      out_shape=out_shape,
      scratch_shapes=scratch,
  )(*args)
  return res


def kernel(x, edge_index, batch, params):
  src = edge_index[0]
  dst = edge_index[1]
  pad = E_PAD - E
  pad_src = jnp.arange(pad, dtype=jnp.int32) % N
  pad_dst = N + (jnp.arange(pad, dtype=jnp.int32) % (N_PAD - N))
  srcw = jnp.concatenate([src, pad_src]).reshape(NW, KC, CH)
  dstw = jnp.concatenate([dst, pad_dst]).reshape(NW, KC, CH)
  zpad = jnp.zeros((RPT, D), jnp.float32)
  batch3 = batch.reshape(NB, 1, R)

  row = lambda a: a.reshape(1, D)
  l2w = jnp.zeros((D, D), jnp.float32).at[:, :C].set(params["lin2_W"])
  l2b = jnp.zeros((1, D), jnp.float32).at[0, :C].set(params["lin2_b"])

  h = x
  for li in range(len(params["convs"])):
    p = params["convs"][li]
    agg = _segsum_sc(h, srcw, dstw, zpad)
    if li < len(params["convs"]) - 1:
      (h,) = _gin_layer_tc(agg, p["W1"], row(p["b1"]), p["W2"],
                           row(p["b2"]), row(p["gamma"]), row(p["beta"]))
    else:
      embeds, out_pad = _gin_layer_tc(
          agg, p["W1"], row(p["b1"]), p["W2"], row(p["b2"]),
          row(p["gamma"]), row(p["beta"]), batch3,
          params["lin1_W"], row(params["lin1_b"]), l2w, l2b)
  return (out_pad[:, :C], embeds)
